# Initial kernel scaffold; baseline (speedup 1.0000x reference)
import jax
import jax.numpy as jnp
from jax.experimental import pallas as pl


def kernel(w_mat, b_row, x):
    raise NotImplementedError("write your pallas kernel here")


# speedup vs baseline: 3.9634x; 3.9634x over previous
import jax
import jax.numpy as jnp
from jax.experimental import pallas as pl
from jax.experimental.pallas import tpu as pltpu

_K = 9
_G = 6
_C = 256
_N = 256
_BB = 8
_CAPS = 8


def _conv_body(x_ref, w_ref, b_ref, o_ref):
    bb = x_ref.shape[4]
    m = _G * _G * bb
    acc = jnp.zeros((m, _N), jnp.float32) + b_ref[...]
    for ky in range(_K):
        py, dy = ky % 2, ky // 2
        for kx in range(_K):
            px, dx = kx % 2, kx // 2
            a = x_ref[py, px, dy:dy + _G, dx:dx + _G, :, :].reshape(m, _C)
            acc += jnp.dot(a, w_ref[ky, kx],
                           preferred_element_type=jnp.float32)
    o_ref[...] = acc.reshape(_G, _G, bb, _N)


def _squash_body(u_ref, o_ref):
    u = u_ref[...]
    sq = jnp.sum(u * u, axis=0, keepdims=True)
    o_ref[...] = u * (sq / ((1.0 + sq) * jnp.sqrt(sq)))


def kernel(w_mat, b_row, x):
    bsz = x.shape[0]
    xq = x.reshape(bsz, _C, 10, 2, 10, 2)
    xq = jnp.transpose(xq, (3, 5, 2, 4, 0, 1)).astype(jnp.bfloat16)
    wq = w_mat.reshape(_K, _K, _C, _N).astype(jnp.bfloat16)

    y = pl.pallas_call(
        _conv_body,
        out_shape=jax.ShapeDtypeStruct((_G, _G, bsz, _N), jnp.float32),
        grid=(bsz // _BB,),
        in_specs=[
            pl.BlockSpec((2, 2, 10, 10, _BB, _C), lambda j: (0, 0, 0, 0, j, 0)),
            pl.BlockSpec((_K, _K, _C, _N), lambda j: (0, 0, 0, 0)),
            pl.BlockSpec((1, _N), lambda j: (0, 0)),
        ],
        out_specs=pl.BlockSpec((_G, _G, _BB, _N), lambda j: (0, 0, j, 0)),
        compiler_params=pltpu.CompilerParams(
            dimension_semantics=("parallel",),
            vmem_limit_bytes=48 * 1024 * 1024,
        ),
    )(xq, wq, b_row)

    u_t = jnp.transpose(y, (2, 3, 0, 1)).reshape(bsz * 1152, _CAPS).T
    l_total = u_t.shape[1]
    n_blk = 8
    out_t = pl.pallas_call(
        _squash_body,
        out_shape=jax.ShapeDtypeStruct((_CAPS, l_total), jnp.float32),
        grid=(n_blk,),
        in_specs=[pl.BlockSpec((_CAPS, l_total // n_blk), lambda j: (0, j))],
        out_specs=pl.BlockSpec((_CAPS, l_total // n_blk), lambda j: (0, j)),
        compiler_params=pltpu.CompilerParams(
            dimension_semantics=("parallel",),
        ),
    )(u_t)
    return out_t.T.reshape(bsz, 1152, _CAPS)


# speedup vs baseline: 4.4315x; 1.1181x over previous
import jax
import jax.numpy as jnp
from jax.experimental import pallas as pl
from jax.experimental.pallas import tpu as pltpu

_K = 9
_G = 6
_C = 256
_N = 256
_BB = 8
_CAPS = 8


def _conv_body(x_ee, x_eo, x_oe, x_oo, w_ref, b_ref, o_ref):
    phases = ((x_ee, x_eo), (x_oe, x_oo))
    bb = x_ee.shape[4]
    m = _G * _G * bb
    acc = jnp.zeros((m, _N), jnp.float32) + b_ref[...]
    for ky in range(_K):
        py, dy = ky % 2, ky // 2
        for kx in range(_K):
            px, dx = kx % 2, kx // 2
            a = phases[py][px][dy:dy + _G, 0, dx:dx + _G, 0, :, :]
            acc += jnp.dot(a.reshape(m, _C), w_ref[ky, kx],
                           preferred_element_type=jnp.float32)
    o_ref[...] = acc.reshape(_G, _G, bb, _N)


def _squash_body(u_ref, o_ref):
    u = u_ref[...]
    sq = jnp.sum(u * u, axis=0, keepdims=True)
    o_ref[...] = u * (sq / ((1.0 + sq) * jnp.sqrt(sq)))


def kernel(w_mat, b_row, x):
    bsz = x.shape[0]
    xt = jnp.transpose(x, (2, 3, 0, 1)).astype(jnp.bfloat16)
    xt = jax.lax.optimization_barrier(xt)
    xq = xt.reshape(10, 2, 10, 2, bsz, _C)
    wq = w_mat.reshape(_K, _K, _C, _N).astype(jnp.bfloat16)

    def _phase_spec(py, px):
        return pl.BlockSpec((10, 1, 10, 1, _BB, _C),
                            lambda j, py=py, px=px: (0, py, 0, px, j, 0))

    y = pl.pallas_call(
        _conv_body,
        out_shape=jax.ShapeDtypeStruct((_G, _G, bsz, _N), jnp.float32),
        grid=(bsz // _BB,),
        in_specs=[
            _phase_spec(0, 0),
            _phase_spec(0, 1),
            _phase_spec(1, 0),
            _phase_spec(1, 1),
            pl.BlockSpec((_K, _K, _C, _N), lambda j: (0, 0, 0, 0)),
            pl.BlockSpec((1, _N), lambda j: (0, 0)),
        ],
        out_specs=pl.BlockSpec((_G, _G, _BB, _N), lambda j: (0, 0, j, 0)),
        compiler_params=pltpu.CompilerParams(
            dimension_semantics=("parallel",),
            vmem_limit_bytes=48 * 1024 * 1024,
        ),
    )(xq, xq, xq, xq, wq, b_row)

    u_t = jnp.transpose(y, (2, 3, 0, 1)).reshape(bsz * 1152, _CAPS).T
    l_total = u_t.shape[1]
    n_blk = 8
    out_t = pl.pallas_call(
        _squash_body,
        out_shape=jax.ShapeDtypeStruct((_CAPS, l_total), jnp.float32),
        grid=(n_blk,),
        in_specs=[pl.BlockSpec((_CAPS, l_total // n_blk), lambda j: (0, j))],
        out_specs=pl.BlockSpec((_CAPS, l_total // n_blk), lambda j: (0, j)),
        compiler_params=pltpu.CompilerParams(
            dimension_semantics=("parallel",),
        ),
    )(u_t)
    return out_t.T.reshape(bsz, 1152, _CAPS)
